# Initial kernel scaffold; baseline (speedup 1.0000x reference)
import jax
import jax.numpy as jnp
from jax.experimental import pallas as pl


def kernel(adj_mats, masks, node_inputs, Wh1, bh1, Wh2, bh2, Wh3, bh3, Wf1, bf1, Wf2, bf2, Wf3, bf3):
    raise NotImplementedError("write your pallas kernel here")


# speedup vs baseline: 11.7935x; 11.7935x over previous
import functools

import jax
import jax.numpy as jnp
from jax import lax
from jax.experimental import pallas as pl
from jax.experimental.pallas import tpu as pltpu
from jax.experimental.pallas import tpu_sc as plsc

_N = 10000
_E = 320000
_DEPTH = 8
_DIN = 128

_NC = 2
_NS = 16
_NW = _NC * _NS
_CHUNK = 128
_KCH = -(-_E // (_NW * _CHUNK))
_EP = _NW * _KCH * _CHUNK
_NP = 10112
_PT = _NP // _NS


def _lk(v):
    return jnp.where(v >= 0, v, 0.01 * v)


def _head_body(ni, wh1, bh1, wh2, bh2, wh3, bh3,
               wf1, bf1, wf2, bf2, wf3, bf3, x_out, y_out):
    x = _lk(jnp.dot(ni[...], wh1[...], preferred_element_type=jnp.float32) + bh1[...])
    x = _lk(jnp.dot(x, wh2[...], preferred_element_type=jnp.float32) + bh2[...])
    x = _lk(jnp.dot(x, wh3[...], preferred_element_type=jnp.float32) + bh3[...])
    x_out[...] = x
    y = _lk(jnp.dot(x, wf1[...], preferred_element_type=jnp.float32) + bf1[...])
    y = _lk(jnp.dot(y, wf2[...], preferred_element_type=jnp.float32) + bf2[...])
    y_out[...] = _lk(jnp.dot(y, wf3[...], preferred_element_type=jnp.float32) + bf3[...])


_head = pl.pallas_call(
    _head_body,
    out_shape=(jax.ShapeDtypeStruct((_NP, 16), jnp.float32),
               jax.ShapeDtypeStruct((_NP, 16), jnp.float32)),
)


def _tail_body(p, xprev, mask, wf1, bf1, wf2, bf2, wf3, bf3, x_out, y_out):
    s = p[0] + p[1]
    t = _lk(jnp.dot(s, wf1[...], preferred_element_type=jnp.float32) + bf1[...])
    t = _lk(jnp.dot(t, wf2[...], preferred_element_type=jnp.float32) + bf2[...])
    t = _lk(jnp.dot(t, wf3[...], preferred_element_type=jnp.float32) + bf3[...])
    xn = xprev[...] + mask[...] * t
    x_out[...] = xn
    y = _lk(jnp.dot(xn, wf1[...], preferred_element_type=jnp.float32) + bf1[...])
    y = _lk(jnp.dot(y, wf2[...], preferred_element_type=jnp.float32) + bf2[...])
    y_out[...] = _lk(jnp.dot(y, wf3[...], preferred_element_type=jnp.float32) + bf3[...])


_tail = pl.pallas_call(
    _tail_body,
    out_shape=(jax.ShapeDtypeStruct((_NP, 16), jnp.float32),
               jax.ShapeDtypeStruct((_NP, 16), jnp.float32)),
)


def _spmm_body(y_hbm, cols_hbm, rows_hbm, out_hbm, cols_v, rows_v, g_v, z_v, acc):
    c = lax.axis_index("c")
    s = lax.axis_index("s")
    wid = s * _NC + c

    def zb(i, carry):
        z_v[i] = jnp.zeros((16,), jnp.float32)
        return carry

    lax.fori_loop(0, _PT, zb, 0)
    pltpu.sync_copy(z_v, acc.at[pl.ds(s * _PT, _PT)])
    pltpu.sync_copy(cols_hbm.at[wid], cols_v)
    pltpu.sync_copy(rows_hbm.at[wid], rows_v)
    plsc.subcore_barrier()

    def body(j, carry):
        pltpu.sync_copy(y_hbm.at[cols_v.at[j]], g_v)
        pltpu.sync_copy(g_v, acc.at[rows_v.at[j]], add=True)
        return carry

    lax.fori_loop(0, _KCH, body, 0)
    plsc.subcore_barrier()
    pltpu.sync_copy(acc.at[pl.ds(s * _PT, _PT)],
                    out_hbm.at[c, pl.ds(s * _PT, _PT)])


@functools.lru_cache(maxsize=1)
def _make_spmm():
    return pl.kernel(
        _spmm_body,
        out_type=jax.ShapeDtypeStruct((_NC, _NP, 16), jnp.float32),
        mesh=plsc.VectorSubcoreMesh(core_axis_name="c", subcore_axis_name="s"),
        compiler_params=pltpu.CompilerParams(use_tc_tiling_on_sc=False),
        scratch_types=[
            pltpu.VMEM((_KCH, _CHUNK), jnp.int32),
            pltpu.VMEM((_KCH, _CHUNK), jnp.int32),
            pltpu.VMEM((_CHUNK, 16), jnp.float32),
            pltpu.VMEM((_PT, 16), jnp.float32),
            pltpu.VMEM_SHARED((_NP, 16), jnp.float32),
        ],
    )


def kernel(adj_mats, masks, node_inputs, Wh1, bh1, Wh2, bh2, Wh3, bh3,
           Wf1, bf1, Wf2, bf2, Wf3, bf3):
    ni = jnp.pad(node_inputs, ((0, _NP - _N), (0, 0)))
    rows = adj_mats[:, 0, :].astype(jnp.int32)
    cols = adj_mats[:, 1, :].astype(jnp.int32)
    pad = _EP - _E
    rows = jnp.pad(rows, ((0, 0), (0, pad)), constant_values=_N)
    cols = jnp.pad(cols, ((0, 0), (0, pad)))
    rows = rows.reshape(_DEPTH, _NW, _KCH, _CHUNK)
    cols = cols.reshape(_DEPTH, _NW, _KCH, _CHUNK)
    masks_p = jnp.pad(masks, ((0, 0), (0, _NP - _N), (0, 0)))

    def b(v):
        return v.reshape(1, -1)

    spmm = _make_spmm()
    x, y = _head(ni, Wh1, b(bh1), Wh2, b(bh2), Wh3, b(bh3),
                 Wf1, b(bf1), Wf2, b(bf2), Wf3, b(bf3))
    for d in range(_DEPTH):
        p = spmm(y, cols[d], rows[d])
        x, y = _tail(p, x, masks_p[d], Wf1, b(bf1), Wf2, b(bf2), Wf3, b(bf3))
    return x[:_N]


# speedup vs baseline: 13.1386x; 1.1141x over previous
import functools

import jax
import jax.numpy as jnp
from jax import lax
from jax.experimental import pallas as pl
from jax.experimental.pallas import tpu as pltpu
from jax.experimental.pallas import tpu_sc as plsc

_N = 10000
_E = 320000
_DEPTH = 8
_DIN = 128

_NC = 2
_NS = 16
_NW = _NC * _NS
_CHUNK = 128
_KB = 4
_KCH = 80
_TB = _KCH // _KB
_EP = _NW * _KCH * _CHUNK
_NP = 10112
_PT = _NP // _NS


def _lk(v):
    return jnp.where(v >= 0, v, 0.01 * v)


def _head_body(ni, wh1, bh1, wh2, bh2, wh3, bh3,
               wf1, bf1, wf2, bf2, wf3, bf3, x_out, y_out):
    x = _lk(jnp.dot(ni[...], wh1[...], preferred_element_type=jnp.float32) + bh1[...])
    x = _lk(jnp.dot(x, wh2[...], preferred_element_type=jnp.float32) + bh2[...])
    x = _lk(jnp.dot(x, wh3[...], preferred_element_type=jnp.float32) + bh3[...])
    x_out[...] = x
    y = _lk(jnp.dot(x, wf1[...], preferred_element_type=jnp.float32) + bf1[...])
    y = _lk(jnp.dot(y, wf2[...], preferred_element_type=jnp.float32) + bf2[...])
    y_out[...] = _lk(jnp.dot(y, wf3[...], preferred_element_type=jnp.float32) + bf3[...])


_head = pl.pallas_call(
    _head_body,
    out_shape=(jax.ShapeDtypeStruct((_NP, 16), jnp.float32),
               jax.ShapeDtypeStruct((_NP, 16), jnp.float32)),
)


def _tail_body(p, xprev, mask, wf1, bf1, wf2, bf2, wf3, bf3, x_out, y_out):
    s = p[0] + p[1]
    t = _lk(jnp.dot(s, wf1[...], preferred_element_type=jnp.float32) + bf1[...])
    t = _lk(jnp.dot(t, wf2[...], preferred_element_type=jnp.float32) + bf2[...])
    t = _lk(jnp.dot(t, wf3[...], preferred_element_type=jnp.float32) + bf3[...])
    xn = xprev[...] + mask[...] * t
    x_out[...] = xn
    y = _lk(jnp.dot(xn, wf1[...], preferred_element_type=jnp.float32) + bf1[...])
    y = _lk(jnp.dot(y, wf2[...], preferred_element_type=jnp.float32) + bf2[...])
    y_out[...] = _lk(jnp.dot(y, wf3[...], preferred_element_type=jnp.float32) + bf3[...])


_tail = pl.pallas_call(
    _tail_body,
    out_shape=(jax.ShapeDtypeStruct((_NP, 16), jnp.float32),
               jax.ShapeDtypeStruct((_NP, 16), jnp.float32)),
)


def _spmm_body(y_hbm, cols_hbm, rows_hbm, out_hbm, cols_v, rows_v, g_v, z_v, acc,
               gsem0, gsem1, ssem0, ssem1):
    gsem = (gsem0, gsem1)
    ssem = (ssem0, ssem1)
    c = lax.axis_index("c")
    s = lax.axis_index("s")
    wid = s * _NC + c

    def zb(i, carry):
        z_v[i] = jnp.zeros((16,), jnp.float32)
        return carry

    lax.fori_loop(0, _PT, zb, 0)
    pltpu.sync_copy(z_v, acc.at[pl.ds(s * _PT, _PT)])
    pltpu.sync_copy(cols_hbm.at[wid], cols_v)
    pltpu.sync_copy(rows_hbm.at[wid], rows_v)
    plsc.subcore_barrier()

    for u in range(_KB):
        pltpu.async_copy(y_hbm.at[cols_v.at[u]], g_v.at[0, u], gsem[0])

    def body(i, carry):
        for bank in range(2):
            t = 2 * i + bank
            other = 1 - bank
            for u in range(_KB):
                pltpu.make_async_copy(
                    y_hbm.at[cols_v.at[t * _KB + u]], g_v.at[bank, u],
                    gsem[bank]).wait()
            for u in range(_KB):
                pltpu.async_copy(g_v.at[bank, u],
                                 acc.at[rows_v.at[t * _KB + u]],
                                 ssem[bank], add=True)

            @pl.when(t > 0)
            def _():
                for u in range(_KB):
                    pltpu.make_async_copy(g_v.at[other, u],
                                          acc.at[rows_v.at[u]],
                                          ssem[other]).wait()

            @pl.when(t + 1 < _TB)
            def _():
                for u in range(_KB):
                    pltpu.async_copy(
                        y_hbm.at[cols_v.at[(t + 1) * _KB + u]],
                        g_v.at[other, u], gsem[other])
        return carry

    lax.fori_loop(0, _TB // 2, body, 0)
    for u in range(_KB):
        pltpu.make_async_copy(g_v.at[1, u], acc.at[rows_v.at[u]],
                              ssem[1]).wait()
    plsc.subcore_barrier()
    pltpu.sync_copy(acc.at[pl.ds(s * _PT, _PT)],
                    out_hbm.at[c, pl.ds(s * _PT, _PT)])


@functools.lru_cache(maxsize=1)
def _make_spmm():
    return pl.kernel(
        _spmm_body,
        out_type=jax.ShapeDtypeStruct((_NC, _NP, 16), jnp.float32),
        mesh=plsc.VectorSubcoreMesh(core_axis_name="c", subcore_axis_name="s"),
        compiler_params=pltpu.CompilerParams(use_tc_tiling_on_sc=False),
        scratch_types=[
            pltpu.VMEM((_KCH, _CHUNK), jnp.int32),
            pltpu.VMEM((_KCH, _CHUNK), jnp.int32),
            pltpu.VMEM((2, _KB, _CHUNK, 16), jnp.float32),
            pltpu.VMEM((_PT, 16), jnp.float32),
            pltpu.VMEM_SHARED((_NP, 16), jnp.float32),
            pltpu.SemaphoreType.DMA,
            pltpu.SemaphoreType.DMA,
            pltpu.SemaphoreType.DMA,
            pltpu.SemaphoreType.DMA,
        ],
    )


def kernel(adj_mats, masks, node_inputs, Wh1, bh1, Wh2, bh2, Wh3, bh3,
           Wf1, bf1, Wf2, bf2, Wf3, bf3):
    ni = jnp.pad(node_inputs, ((0, _NP - _N), (0, 0)))
    rows = adj_mats[:, 0, :].astype(jnp.int32)
    cols = adj_mats[:, 1, :].astype(jnp.int32)
    pad = _EP - _E
    rows = jnp.pad(rows, ((0, 0), (0, pad)), constant_values=_N)
    cols = jnp.pad(cols, ((0, 0), (0, pad)))
    rows = rows.reshape(_DEPTH, _NW, _KCH, _CHUNK)
    cols = cols.reshape(_DEPTH, _NW, _KCH, _CHUNK)
    masks_p = jnp.pad(masks, ((0, 0), (0, _NP - _N), (0, 0)))

    def b(v):
        return v.reshape(1, -1)

    spmm = _make_spmm()
    x, y = _head(ni, Wh1, b(bh1), Wh2, b(bh2), Wh3, b(bh3),
                 Wf1, b(bf1), Wf2, b(bf2), Wf3, b(bf3))
    for d in range(_DEPTH):
        p = spmm(y, cols[d], rows[d])
        x, y = _tail(p, x, masks_p[d], Wf1, b(bf1), Wf2, b(bf2), Wf3, b(bf3))
    return x[:_N]


# speedup vs baseline: 13.8105x; 1.0511x over previous
import functools

import jax
import jax.numpy as jnp
from jax import lax
from jax.experimental import pallas as pl
from jax.experimental.pallas import tpu as pltpu
from jax.experimental.pallas import tpu_sc as plsc

_N = 10000
_E = 320000
_DEPTH = 8
_DIN = 128

_NC = 2
_NS = 16
_NW = _NC * _NS
_CHUNK = 256
_KB = 4
_KCH = 40
_TB = _KCH // _KB
_EP = _NW * _KCH * _CHUNK
_NP = 10112
_PT = _NP // _NS


def _lk(v):
    return jnp.where(v >= 0, v, 0.01 * v)


def _head_body(ni, wh1, bh1, wh2, bh2, wh3, bh3,
               wf1, bf1, wf2, bf2, wf3, bf3, x_out, y_out):
    x = _lk(jnp.dot(ni[...], wh1[...], preferred_element_type=jnp.float32) + bh1[...])
    x = _lk(jnp.dot(x, wh2[...], preferred_element_type=jnp.float32) + bh2[...])
    x = _lk(jnp.dot(x, wh3[...], preferred_element_type=jnp.float32) + bh3[...])
    x_out[...] = x
    y = _lk(jnp.dot(x, wf1[...], preferred_element_type=jnp.float32) + bf1[...])
    y = _lk(jnp.dot(y, wf2[...], preferred_element_type=jnp.float32) + bf2[...])
    y_out[...] = _lk(jnp.dot(y, wf3[...], preferred_element_type=jnp.float32) + bf3[...])


_head = pl.pallas_call(
    _head_body,
    out_shape=(jax.ShapeDtypeStruct((_NP, 16), jnp.float32),
               jax.ShapeDtypeStruct((_NP, 16), jnp.float32)),
)


def _tail_body(p, xprev, mask, wf1, bf1, wf2, bf2, wf3, bf3, x_out, y_out):
    s = p[0] + p[1]
    t = _lk(jnp.dot(s, wf1[...], preferred_element_type=jnp.float32) + bf1[...])
    t = _lk(jnp.dot(t, wf2[...], preferred_element_type=jnp.float32) + bf2[...])
    t = _lk(jnp.dot(t, wf3[...], preferred_element_type=jnp.float32) + bf3[...])
    xn = xprev[...] + mask[...] * t
    x_out[...] = xn
    y = _lk(jnp.dot(xn, wf1[...], preferred_element_type=jnp.float32) + bf1[...])
    y = _lk(jnp.dot(y, wf2[...], preferred_element_type=jnp.float32) + bf2[...])
    y_out[...] = _lk(jnp.dot(y, wf3[...], preferred_element_type=jnp.float32) + bf3[...])


_tail = pl.pallas_call(
    _tail_body,
    out_shape=(jax.ShapeDtypeStruct((_NP, 16), jnp.float32),
               jax.ShapeDtypeStruct((_NP, 16), jnp.float32)),
)


def _spmm_body(zeros_hbm, y_hbm, cols_hbm, rows_hbm, out_hbm, cols_v, rows_v,
               g_v, acc, gsem0, gsem1, ssem0, ssem1):
    gsem = (gsem0, gsem1)
    ssem = (ssem0, ssem1)
    c = lax.axis_index("c")
    s = lax.axis_index("s")
    wid = s * _NC + c

    pltpu.sync_copy(zeros_hbm, acc.at[pl.ds(s * _PT, _PT)])
    pltpu.sync_copy(cols_hbm.at[wid], cols_v)
    pltpu.sync_copy(rows_hbm.at[wid], rows_v)
    plsc.subcore_barrier()

    for u in range(_KB):
        pltpu.async_copy(y_hbm.at[cols_v.at[u]], g_v.at[0, u], gsem[0])

    def body(i, carry):
        for bank in range(2):
            t = 2 * i + bank
            other = 1 - bank
            for u in range(_KB):
                pltpu.make_async_copy(
                    y_hbm.at[cols_v.at[t * _KB + u]], g_v.at[bank, u],
                    gsem[bank]).wait()
            for u in range(_KB):
                pltpu.async_copy(g_v.at[bank, u],
                                 acc.at[rows_v.at[t * _KB + u]],
                                 ssem[bank], add=True)

            @pl.when(t > 0)
            def _():
                for u in range(_KB):
                    pltpu.make_async_copy(g_v.at[other, u],
                                          acc.at[rows_v.at[u]],
                                          ssem[other]).wait()

            @pl.when(t + 1 < _TB)
            def _():
                for u in range(_KB):
                    pltpu.async_copy(
                        y_hbm.at[cols_v.at[(t + 1) * _KB + u]],
                        g_v.at[other, u], gsem[other])
        return carry

    lax.fori_loop(0, _TB // 2, body, 0)
    for u in range(_KB):
        pltpu.make_async_copy(g_v.at[1, u], acc.at[rows_v.at[u]],
                              ssem[1]).wait()
    plsc.subcore_barrier()
    pltpu.sync_copy(acc.at[pl.ds(s * _PT, _PT)],
                    out_hbm.at[c, pl.ds(s * _PT, _PT)])


@functools.lru_cache(maxsize=1)
def _make_spmm():
    return pl.kernel(
        _spmm_body,
        out_type=jax.ShapeDtypeStruct((_NC, _NP, 16), jnp.float32),
        mesh=plsc.VectorSubcoreMesh(core_axis_name="c", subcore_axis_name="s"),
        compiler_params=pltpu.CompilerParams(use_tc_tiling_on_sc=False),
        scratch_types=[
            pltpu.VMEM((_KCH, _CHUNK), jnp.int32),
            pltpu.VMEM((_KCH, _CHUNK), jnp.int32),
            pltpu.VMEM((2, _KB, _CHUNK, 16), jnp.float32),
            pltpu.VMEM_SHARED((_NP, 16), jnp.float32),
            pltpu.SemaphoreType.DMA,
            pltpu.SemaphoreType.DMA,
            pltpu.SemaphoreType.DMA,
            pltpu.SemaphoreType.DMA,
        ],
    )


def kernel(adj_mats, masks, node_inputs, Wh1, bh1, Wh2, bh2, Wh3, bh3,
           Wf1, bf1, Wf2, bf2, Wf3, bf3):
    ni = jnp.pad(node_inputs, ((0, _NP - _N), (0, 0)))
    rows = adj_mats[:, 0, :].astype(jnp.int32)
    cols = adj_mats[:, 1, :].astype(jnp.int32)
    pad = _EP - _E
    rows = jnp.pad(rows, ((0, 0), (0, pad)), constant_values=_N)
    cols = jnp.pad(cols, ((0, 0), (0, pad)))
    rows = rows.reshape(_DEPTH, _NW, _KCH, _CHUNK)
    cols = cols.reshape(_DEPTH, _NW, _KCH, _CHUNK)
    masks_p = jnp.pad(masks, ((0, 0), (0, _NP - _N), (0, 0)))

    def b(v):
        return v.reshape(1, -1)

    spmm = _make_spmm()
    zeros = jnp.zeros((_PT, 16), jnp.float32)
    x, y = _head(ni, Wh1, b(bh1), Wh2, b(bh2), Wh3, b(bh3),
                 Wf1, b(bf1), Wf2, b(bf2), Wf3, b(bf3))
    for d in range(_DEPTH):
        p = spmm(zeros, y, cols[d], rows[d])
        x, y = _tail(p, x, masks_p[d], Wf1, b(bf1), Wf2, b(bf2), Wf3, b(bf3))
    return x[:_N]


# speedup vs baseline: 16.9682x; 1.2286x over previous
import functools

import jax
import jax.numpy as jnp
from jax import lax
from jax.experimental import pallas as pl
from jax.experimental.pallas import tpu as pltpu
from jax.experimental.pallas import tpu_sc as plsc

_N = 10000
_E = 320000
_DEPTH = 8
_DIN = 128

_NC = 2
_NS = 16
_NW = _NC * _NS
_CHUNK = 256
_KB = 4
_KCH = 40
_TB = _KCH // _KB
_EP = _NW * _KCH * _CHUNK
_NP = 10112
_PT = _NP // _NS
_GP = _NP // 8


def _lk(v):
    return jnp.where(v >= 0, v, 0.01 * v)


def _head_body(ni, wh1, bh1, wh2, bh2, wh3, bh3,
               wf1, bf1, wf2, bf2, wf3, bf3, x_out, y_out):
    x = _lk(jnp.dot(ni[...], wh1[...], preferred_element_type=jnp.float32) + bh1[...])
    x = _lk(jnp.dot(x, wh2[...], preferred_element_type=jnp.float32) + bh2[...])
    x = _lk(jnp.dot(x, wh3[...], preferred_element_type=jnp.float32) + bh3[...])
    x_out[...] = x
    y = _lk(jnp.dot(x, wf1[...], preferred_element_type=jnp.float32) + bf1[...])
    y = _lk(jnp.dot(y, wf2[...], preferred_element_type=jnp.float32) + bf2[...])
    y_out[...] = _lk(jnp.dot(y, wf3[...], preferred_element_type=jnp.float32) + bf3[...])


_head = pl.pallas_call(
    _head_body,
    out_shape=(jax.ShapeDtypeStruct((_NP, 16), jnp.float32),
               jax.ShapeDtypeStruct((_NP, 16), jnp.float32)),
)


def _tail_body(p, xprev, mask, wb1, bb1, wb2, bb2, wb3, bb3, x_out, y_out):
    s = p[0] + p[1]
    t = _lk(jnp.dot(s, wb1[...], preferred_element_type=jnp.float32) + bb1[...])
    t = _lk(jnp.dot(t, wb2[...], preferred_element_type=jnp.float32) + bb2[...])
    t = _lk(jnp.dot(t, wb3[...], preferred_element_type=jnp.float32) + bb3[...])
    xn = xprev[...] + mask[...] * t
    x_out[...] = xn
    y = _lk(jnp.dot(xn, wb1[...], preferred_element_type=jnp.float32) + bb1[...])
    y = _lk(jnp.dot(y, wb2[...], preferred_element_type=jnp.float32) + bb2[...])
    y_out[...] = _lk(jnp.dot(y, wb3[...], preferred_element_type=jnp.float32) + bb3[...])


_tail = pl.pallas_call(
    _tail_body,
    out_shape=(jax.ShapeDtypeStruct((_GP, 128), jnp.float32),
               jax.ShapeDtypeStruct((_GP, 128), jnp.float32)),
)


def _spmm_body(zeros_hbm, y_hbm, cols_hbm, rows_hbm, out_hbm, cols_v, rows_v,
               g_v, acc, gsem0, gsem1, ssem0, ssem1):
    gsem = (gsem0, gsem1)
    ssem = (ssem0, ssem1)
    c = lax.axis_index("c")
    s = lax.axis_index("s")
    wid = s * _NC + c

    pltpu.sync_copy(zeros_hbm, acc.at[pl.ds(s * _PT, _PT)])
    pltpu.sync_copy(cols_hbm.at[wid], cols_v)
    pltpu.sync_copy(rows_hbm.at[wid], rows_v)
    plsc.subcore_barrier()

    for u in range(_KB):
        pltpu.async_copy(y_hbm.at[cols_v.at[u]], g_v.at[0, u], gsem[0])

    def body(i, carry):
        for bank in range(2):
            t = 2 * i + bank
            other = 1 - bank
            for u in range(_KB):
                pltpu.make_async_copy(
                    y_hbm.at[cols_v.at[t * _KB + u]], g_v.at[bank, u],
                    gsem[bank]).wait()
            for u in range(_KB):
                pltpu.async_copy(g_v.at[bank, u],
                                 acc.at[rows_v.at[t * _KB + u]],
                                 ssem[bank], add=True)

            @pl.when(t > 0)
            def _():
                for u in range(_KB):
                    pltpu.make_async_copy(g_v.at[other, u],
                                          acc.at[rows_v.at[u]],
                                          ssem[other]).wait()

            @pl.when(t + 1 < _TB)
            def _():
                for u in range(_KB):
                    pltpu.async_copy(
                        y_hbm.at[cols_v.at[(t + 1) * _KB + u]],
                        g_v.at[other, u], gsem[other])
        return carry

    lax.fori_loop(0, _TB // 2, body, 0)
    for u in range(_KB):
        pltpu.make_async_copy(g_v.at[1, u], acc.at[rows_v.at[u]],
                              ssem[1]).wait()
    plsc.subcore_barrier()
    pltpu.sync_copy(acc.at[pl.ds(s * _PT, _PT)],
                    out_hbm.at[c, pl.ds(s * _PT, _PT)])


@functools.lru_cache(maxsize=1)
def _make_spmm():
    return pl.kernel(
        _spmm_body,
        out_type=jax.ShapeDtypeStruct((_NC, _NP, 16), jnp.float32),
        mesh=plsc.VectorSubcoreMesh(core_axis_name="c", subcore_axis_name="s"),
        compiler_params=pltpu.CompilerParams(use_tc_tiling_on_sc=False),
        scratch_types=[
            pltpu.VMEM((_KCH, _CHUNK), jnp.int32),
            pltpu.VMEM((_KCH, _CHUNK), jnp.int32),
            pltpu.VMEM((2, _KB, _CHUNK, 16), jnp.float32),
            pltpu.VMEM_SHARED((_NP, 16), jnp.float32),
            pltpu.SemaphoreType.DMA,
            pltpu.SemaphoreType.DMA,
            pltpu.SemaphoreType.DMA,
            pltpu.SemaphoreType.DMA,
        ],
    )


def kernel(adj_mats, masks, node_inputs, Wh1, bh1, Wh2, bh2, Wh3, bh3,
           Wf1, bf1, Wf2, bf2, Wf3, bf3):
    ni = jnp.pad(node_inputs, ((0, _NP - _N), (0, 0)))
    rows = adj_mats[:, 0, :].astype(jnp.int32)
    cols = adj_mats[:, 1, :].astype(jnp.int32)
    pad = _EP - _E
    rows = jnp.pad(rows, ((0, 0), (0, pad)), constant_values=_N)
    cols = jnp.pad(cols, ((0, 0), (0, pad)))
    rows = rows.reshape(_DEPTH, _NW, _KCH, _CHUNK)
    cols = cols.reshape(_DEPTH, _NW, _KCH, _CHUNK)
    masks_p = jnp.pad(masks, ((0, 0), (0, _NP - _N), (0, 0)))

    def b(v):
        return v.reshape(1, -1)

    eye8 = jnp.eye(8, dtype=jnp.float32)
    wb1 = jnp.kron(eye8, Wf1)
    wb2 = jnp.kron(eye8, Wf2)
    wb3 = jnp.kron(eye8, Wf3)
    bb1 = jnp.tile(bf1, 8).reshape(1, -1)
    bb2 = jnp.tile(bf2, 8).reshape(1, -1)
    bb3 = jnp.tile(bf3, 8).reshape(1, -1)
    masks_pk = jnp.broadcast_to(masks_p, (_DEPTH, _NP, 16)).reshape(
        _DEPTH, _GP, 128)

    spmm = _make_spmm()
    zeros = jnp.zeros((_PT, 16), jnp.float32)
    x, y = _head(ni, Wh1, b(bh1), Wh2, b(bh2), Wh3, b(bh3),
                 Wf1, b(bf1), Wf2, b(bf2), Wf3, b(bf3))
    x = x.reshape(_GP, 128)
    y = y.reshape(_GP, 128)
    for d in range(_DEPTH):
        p = spmm(zeros, y.reshape(_NP, 16), cols[d], rows[d])
        x, y = _tail(p.reshape(_NC, _GP, 128), x, masks_pk[d],
                     wb1, bb1, wb2, bb2, wb3, bb3)
    return x.reshape(_NP, 16)[:_N]


# speedup vs baseline: 30.1168x; 1.7749x over previous
import functools

import jax
import jax.numpy as jnp
from jax import lax
from jax.experimental import pallas as pl
from jax.experimental.pallas import tpu as pltpu
from jax.experimental.pallas import tpu_sc as plsc

_N = 10000
_E = 320000
_DEPTH = 8
_DIN = 128

_NC = 2
_NS = 16
_NW = _NC * _NS
_CHUNK = 256
_KB = 4
_KCH = 40
_TB = _KCH // _KB
_EP = _NW * _KCH * _CHUNK
_NP = 10112
_PT = _NP // _NS
_GP = _NP // 8


def _lk(v):
    return jnp.where(v >= 0, v, 0.01 * v)


def _head_body(ni, wh1, bh1, wh2, bh2, wh3, bh3,
               wf1, bf1, wf2, bf2, wf3, bf3, x_out, y_out):
    x = _lk(jnp.dot(ni[...], wh1[...], preferred_element_type=jnp.float32) + bh1[...])
    x = _lk(jnp.dot(x, wh2[...], preferred_element_type=jnp.float32) + bh2[...])
    x = _lk(jnp.dot(x, wh3[...], preferred_element_type=jnp.float32) + bh3[...])
    x_out[...] = x
    y = _lk(jnp.dot(x, wf1[...], preferred_element_type=jnp.float32) + bf1[...])
    y = _lk(jnp.dot(y, wf2[...], preferred_element_type=jnp.float32) + bf2[...])
    y_out[...] = _lk(jnp.dot(y, wf3[...], preferred_element_type=jnp.float32) + bf3[...])


_head = pl.pallas_call(
    _head_body,
    out_shape=(jax.ShapeDtypeStruct((_NP, 16), jnp.float32),
               jax.ShapeDtypeStruct((_NP, 16), jnp.float32)),
)


def _tail_body(p, xprev, mask, wb1, bb1, wb2, bb2, wb3, bb3, x_out, y_out):
    s = p[0] + p[1]
    t = _lk(jnp.dot(s, wb1[...], preferred_element_type=jnp.float32) + bb1[...])
    t = _lk(jnp.dot(t, wb2[...], preferred_element_type=jnp.float32) + bb2[...])
    t = _lk(jnp.dot(t, wb3[...], preferred_element_type=jnp.float32) + bb3[...])
    xn = xprev[...] + mask[...] * t
    x_out[...] = xn
    y = _lk(jnp.dot(xn, wb1[...], preferred_element_type=jnp.float32) + bb1[...])
    y = _lk(jnp.dot(y, wb2[...], preferred_element_type=jnp.float32) + bb2[...])
    y_out[...] = _lk(jnp.dot(y, wb3[...], preferred_element_type=jnp.float32) + bb3[...])


_tail = pl.pallas_call(
    _tail_body,
    out_shape=(jax.ShapeDtypeStruct((_GP, 128), jnp.float32),
               jax.ShapeDtypeStruct((_GP, 128), jnp.float32)),
)


def _spmm_body(zeros_hbm, y_hbm, cols_hbm, rows_hbm, out_hbm, cols_v, rows_v,
               g_v, acc, y_sp, gsem0, gsem1, ssem0, ssem1):
    gsem = (gsem0, gsem1)
    ssem = (ssem0, ssem1)
    c = lax.axis_index("c")
    s = lax.axis_index("s")
    wid = s * _NC + c

    pltpu.sync_copy(zeros_hbm, acc.at[pl.ds(s * _PT, _PT)])
    pltpu.sync_copy(y_hbm.at[pl.ds(s * _PT, _PT)],
                    y_sp.at[pl.ds(s * _PT, _PT)])
    pltpu.sync_copy(cols_hbm.at[wid], cols_v)
    pltpu.sync_copy(rows_hbm.at[wid], rows_v)
    plsc.subcore_barrier()

    for u in range(_KB):
        pltpu.async_copy(y_sp.at[cols_v.at[u]], g_v.at[0, u], gsem[0])

    def body(i, carry):
        for bank in range(2):
            t = 2 * i + bank
            other = 1 - bank
            for u in range(_KB):
                pltpu.make_async_copy(
                    y_sp.at[cols_v.at[t * _KB + u]], g_v.at[bank, u],
                    gsem[bank]).wait()
            for u in range(_KB):
                pltpu.async_copy(g_v.at[bank, u],
                                 acc.at[rows_v.at[t * _KB + u]],
                                 ssem[bank], add=True)

            @pl.when(t > 0)
            def _():
                for u in range(_KB):
                    pltpu.make_async_copy(g_v.at[other, u],
                                          acc.at[rows_v.at[u]],
                                          ssem[other]).wait()

            @pl.when(t + 1 < _TB)
            def _():
                for u in range(_KB):
                    pltpu.async_copy(
                        y_sp.at[cols_v.at[(t + 1) * _KB + u]],
                        g_v.at[other, u], gsem[other])
        return carry

    lax.fori_loop(0, _TB // 2, body, 0)
    for u in range(_KB):
        pltpu.make_async_copy(g_v.at[1, u], acc.at[rows_v.at[u]],
                              ssem[1]).wait()
    plsc.subcore_barrier()
    pltpu.sync_copy(acc.at[pl.ds(s * _PT, _PT)],
                    out_hbm.at[c, pl.ds(s * _PT, _PT)])


@functools.lru_cache(maxsize=1)
def _make_spmm():
    return pl.kernel(
        _spmm_body,
        out_type=jax.ShapeDtypeStruct((_NC, _NP, 16), jnp.float32),
        mesh=plsc.VectorSubcoreMesh(core_axis_name="c", subcore_axis_name="s"),
        compiler_params=pltpu.CompilerParams(use_tc_tiling_on_sc=False),
        scratch_types=[
            pltpu.VMEM((_KCH, _CHUNK), jnp.int32),
            pltpu.VMEM((_KCH, _CHUNK), jnp.int32),
            pltpu.VMEM((2, _KB, _CHUNK, 16), jnp.float32),
            pltpu.VMEM_SHARED((_NP, 16), jnp.float32),
            pltpu.VMEM_SHARED((_NP, 16), jnp.float32),
            pltpu.SemaphoreType.DMA,
            pltpu.SemaphoreType.DMA,
            pltpu.SemaphoreType.DMA,
            pltpu.SemaphoreType.DMA,
        ],
    )


def kernel(adj_mats, masks, node_inputs, Wh1, bh1, Wh2, bh2, Wh3, bh3,
           Wf1, bf1, Wf2, bf2, Wf3, bf3):
    ni = jnp.pad(node_inputs, ((0, _NP - _N), (0, 0)))
    rows = adj_mats[:, 0, :].astype(jnp.int32)
    cols = adj_mats[:, 1, :].astype(jnp.int32)
    pad = _EP - _E
    rows = jnp.pad(rows, ((0, 0), (0, pad)), constant_values=_N)
    cols = jnp.pad(cols, ((0, 0), (0, pad)))
    rows = rows.reshape(_DEPTH, _NW, _KCH, _CHUNK)
    cols = cols.reshape(_DEPTH, _NW, _KCH, _CHUNK)
    masks_p = jnp.pad(masks, ((0, 0), (0, _NP - _N), (0, 0)))

    def b(v):
        return v.reshape(1, -1)

    eye8 = jnp.eye(8, dtype=jnp.float32)
    wb1 = jnp.kron(eye8, Wf1)
    wb2 = jnp.kron(eye8, Wf2)
    wb3 = jnp.kron(eye8, Wf3)
    bb1 = jnp.tile(bf1, 8).reshape(1, -1)
    bb2 = jnp.tile(bf2, 8).reshape(1, -1)
    bb3 = jnp.tile(bf3, 8).reshape(1, -1)
    masks_pk = jnp.broadcast_to(masks_p, (_DEPTH, _NP, 16)).reshape(
        _DEPTH, _GP, 128)

    spmm = _make_spmm()
    zeros = jnp.zeros((_PT, 16), jnp.float32)
    x, y = _head(ni, Wh1, b(bh1), Wh2, b(bh2), Wh3, b(bh3),
                 Wf1, b(bf1), Wf2, b(bf2), Wf3, b(bf3))
    x = x.reshape(_GP, 128)
    y = y.reshape(_GP, 128)
    for d in range(_DEPTH):
        p = spmm(zeros, y.reshape(_NP, 16), cols[d], rows[d])
        x, y = _tail(p.reshape(_NC, _GP, 128), x, masks_pk[d],
                     wb1, bb1, wb2, bb2, wb3, bb3)
    return x.reshape(_NP, 16)[:_N]


# speedup vs baseline: 32.8450x; 1.0906x over previous
import functools

import jax
import jax.numpy as jnp
from jax import lax
from jax.experimental import pallas as pl
from jax.experimental.pallas import tpu as pltpu
from jax.experimental.pallas import tpu_sc as plsc

_N = 10000
_E = 320000
_DEPTH = 8
_DIN = 128

_NC = 2
_NS = 16
_NW = _NC * _NS
_CHUNK = 128
_KB = 4
_KCH = 80
_TB = _KCH // _KB
_EP = _NW * _KCH * _CHUNK
_NP = 10112
_PT = _NP // _NS
_GP = _NP // 8


def _lk(v):
    return jnp.where(v >= 0, v, 0.01 * v)


def _head_body(ni, wh1, bh1, wh2, bh2, wh3, bh3,
               wf1, bf1, wf2, bf2, wf3, bf3, x_out, y_out):
    x = _lk(jnp.dot(ni[...], wh1[...], preferred_element_type=jnp.float32) + bh1[...])
    x = _lk(jnp.dot(x, wh2[...], preferred_element_type=jnp.float32) + bh2[...])
    x = _lk(jnp.dot(x, wh3[...], preferred_element_type=jnp.float32) + bh3[...])
    x_out[...] = x
    y = _lk(jnp.dot(x, wf1[...], preferred_element_type=jnp.float32) + bf1[...])
    y = _lk(jnp.dot(y, wf2[...], preferred_element_type=jnp.float32) + bf2[...])
    y_out[...] = _lk(jnp.dot(y, wf3[...], preferred_element_type=jnp.float32) + bf3[...])


_head = pl.pallas_call(
    _head_body,
    out_shape=(jax.ShapeDtypeStruct((_NP, 16), jnp.float32),
               jax.ShapeDtypeStruct((_NP, 16), jnp.float32)),
)


def _tail_body(p, xprev, mask, wb1, bb1, wb2, bb2, wb3, bb3, x_out, y_out):
    s = p[0] + p[1]
    t = _lk(jnp.dot(s, wb1[...], preferred_element_type=jnp.float32) + bb1[...])
    t = _lk(jnp.dot(t, wb2[...], preferred_element_type=jnp.float32) + bb2[...])
    t = _lk(jnp.dot(t, wb3[...], preferred_element_type=jnp.float32) + bb3[...])
    xn = xprev[...] + mask[...] * t
    x_out[...] = xn
    y = _lk(jnp.dot(xn, wb1[...], preferred_element_type=jnp.float32) + bb1[...])
    y = _lk(jnp.dot(y, wb2[...], preferred_element_type=jnp.float32) + bb2[...])
    y_out[...] = _lk(jnp.dot(y, wb3[...], preferred_element_type=jnp.float32) + bb3[...])


_tail = pl.pallas_call(
    _tail_body,
    out_shape=(jax.ShapeDtypeStruct((_GP, 128), jnp.float32),
               jax.ShapeDtypeStruct((_GP, 128), jnp.float32)),
)


def _spmm_body(d, zeros_hbm, y_hbm, cols_hbm, rows_hbm, out_hbm, cols_v,
               rows_v, g_v, acc, y_sp, gsem0, gsem1, ssem0, ssem1):
    gsem = (gsem0, gsem1)
    ssem = (ssem0, ssem1)
    c = lax.axis_index("c")
    s = lax.axis_index("s")
    wid = s * _NC + c

    pltpu.sync_copy(zeros_hbm, acc.at[pl.ds(s * _PT, _PT)])
    pltpu.sync_copy(y_hbm.at[pl.ds(s * _PT, _PT)],
                    y_sp.at[pl.ds(s * _PT, _PT)])
    pltpu.sync_copy(cols_hbm.at[d, wid], cols_v)
    pltpu.sync_copy(rows_hbm.at[d, wid], rows_v)
    plsc.subcore_barrier()

    for u in range(_KB):
        pltpu.async_copy(y_sp.at[cols_v.at[u]], g_v.at[0, u], gsem[0])

    def body(i, carry):
        for bank in range(2):
            t = 2 * i + bank
            other = 1 - bank
            for u in range(_KB):
                pltpu.make_async_copy(
                    y_sp.at[cols_v.at[t * _KB + u]], g_v.at[bank, u],
                    gsem[bank]).wait()
            for u in range(_KB):
                pltpu.async_copy(g_v.at[bank, u],
                                 acc.at[rows_v.at[t * _KB + u]],
                                 ssem[bank], add=True)

            @pl.when(t > 0)
            def _():
                for u in range(_KB):
                    pltpu.make_async_copy(g_v.at[other, u],
                                          acc.at[rows_v.at[u]],
                                          ssem[other]).wait()

            @pl.when(t + 1 < _TB)
            def _():
                for u in range(_KB):
                    pltpu.async_copy(
                        y_sp.at[cols_v.at[(t + 1) * _KB + u]],
                        g_v.at[other, u], gsem[other])
        return carry

    lax.fori_loop(0, _TB // 2, body, 0)
    for u in range(_KB):
        pltpu.make_async_copy(g_v.at[1, u], acc.at[rows_v.at[u]],
                              ssem[1]).wait()
    plsc.subcore_barrier()
    pltpu.sync_copy(acc.at[pl.ds(s * _PT, _PT)],
                    out_hbm.at[c, pl.ds(s * _PT, _PT)])


@functools.lru_cache(maxsize=_DEPTH)
def _make_spmm(d):
    return pl.kernel(
        functools.partial(_spmm_body, d),
        out_type=jax.ShapeDtypeStruct((_NC, _NP, 16), jnp.float32),
        mesh=plsc.VectorSubcoreMesh(core_axis_name="c", subcore_axis_name="s"),
        compiler_params=pltpu.CompilerParams(use_tc_tiling_on_sc=False),
        scratch_types=[
            pltpu.VMEM((_KCH, _CHUNK), jnp.int32),
            pltpu.VMEM((_KCH, _CHUNK), jnp.int32),
            pltpu.VMEM((2, _KB, _CHUNK, 16), jnp.float32),
            pltpu.VMEM_SHARED((_NP, 16), jnp.float32),
            pltpu.VMEM_SHARED((_NP, 16), jnp.float32),
            pltpu.SemaphoreType.DMA,
            pltpu.SemaphoreType.DMA,
            pltpu.SemaphoreType.DMA,
            pltpu.SemaphoreType.DMA,
        ],
    )


def kernel(adj_mats, masks, node_inputs, Wh1, bh1, Wh2, bh2, Wh3, bh3,
           Wf1, bf1, Wf2, bf2, Wf3, bf3):
    ni = jnp.pad(node_inputs, ((0, _NP - _N), (0, 0)))
    rows = adj_mats[:, 0, :].astype(jnp.int32)
    cols = adj_mats[:, 1, :].astype(jnp.int32)
    pad = _EP - _E
    rows = jnp.pad(rows, ((0, 0), (0, pad)), constant_values=_N)
    cols = jnp.pad(cols, ((0, 0), (0, pad)))
    rows = rows.reshape(_DEPTH, _NW, _KCH, _CHUNK)
    cols = cols.reshape(_DEPTH, _NW, _KCH, _CHUNK)
    masks_p = jnp.pad(masks, ((0, 0), (0, _NP - _N), (0, 0)))

    def b(v):
        return v.reshape(1, -1)

    eye8 = jnp.eye(8, dtype=jnp.float32)
    wb1 = jnp.kron(eye8, Wf1)
    wb2 = jnp.kron(eye8, Wf2)
    wb3 = jnp.kron(eye8, Wf3)
    bb1 = jnp.tile(bf1, 8).reshape(1, -1)
    bb2 = jnp.tile(bf2, 8).reshape(1, -1)
    bb3 = jnp.tile(bf3, 8).reshape(1, -1)
    masks_pk = jnp.broadcast_to(masks_p, (_DEPTH, _NP, 16)).reshape(
        _DEPTH, _GP, 128)

    zeros = jnp.zeros((_PT, 16), jnp.float32)
    x, y = _head(ni, Wh1, b(bh1), Wh2, b(bh2), Wh3, b(bh3),
                 Wf1, b(bf1), Wf2, b(bf2), Wf3, b(bf3))
    x = x.reshape(_GP, 128)
    y = y.reshape(_GP, 128)
    for d in range(_DEPTH):
        p = _make_spmm(d)(zeros, y.reshape(_NP, 16), cols, rows)
        x, y = _tail(p.reshape(_NC, _GP, 128), x, masks_pk[d],
                     wb1, bb1, wb2, bb2, wb3, bb3)
    return x.reshape(_NP, 16)[:_N]
